# Initial kernel scaffold; baseline (speedup 1.0000x reference)
import functools

import jax
import jax.numpy as jnp
from jax import lax
from jax.experimental import pallas as pl
from jax.experimental.pallas import tpu as pltpu
from jax.experimental.pallas import tpu_sc as plsc

N = 10000
E = 320000
D = 128

NC = 2
NS = 16
NW = NC * NS

CHUNK = 125
EDGES_PER_W = E // NW
NCH_AGG = EDGES_PER_W // CHUNK
EDGES_PER_T = E // NS
NCH_HIST = EDGES_PER_T // CHUNK
ROWS_PER_T = N // NS

_mesh = plsc.VectorSubcoreMesh(core_axis_name="c", subcore_axis_name="s")


def _zero_rows(buf, nrows):
    ncol = buf.shape[1] // 16

    def body(t, _):
        r = t // ncol
        col = t % ncol
        buf[r, pl.ds(col * 16, 16)] = jnp.zeros((16,), jnp.float32)
        return 0

    lax.fori_loop(0, nrows * ncol, body, 0)


@functools.partial(
    pl.kernel,
    out_type=jax.ShapeDtypeStruct((2, N, 16), jnp.float32),
    mesh=_mesh,
    scratch_types=[
        pltpu.VMEM_SHARED((N, 16), jnp.float32),
        pltpu.VMEM((NCH_HIST, CHUNK), jnp.int32),
        pltpu.VMEM((CHUNK, 16), jnp.float32),
        pltpu.VMEM((ROWS_PER_T, 16), jnp.float32),
    ],
)
def _hist_kernel(idx_hbm, hist_hbm, acc, idx_v, ones_v, zero_v):
    c = lax.axis_index("c")
    s = lax.axis_index("s")

    _zero_rows(zero_v, ROWS_PER_T)

    def fill_ones(t, _):
        ones_v[t, :] = jnp.ones((16,), jnp.float32)
        return 0

    lax.fori_loop(0, CHUNK, fill_ones, 0)

    pltpu.sync_copy(zero_v, acc.at[pl.ds(s * ROWS_PER_T, ROWS_PER_T)])
    plsc.subcore_barrier()

    pltpu.sync_copy(idx_hbm.at[c, s], idx_v)

    def body(j, _):
        pltpu.sync_copy(ones_v, acc.at[idx_v.at[j]], add=True)
        return 0

    lax.fori_loop(0, NCH_HIST, body, 0)
    plsc.subcore_barrier()

    pltpu.sync_copy(
        acc.at[pl.ds(s * ROWS_PER_T, ROWS_PER_T)],
        hist_hbm.at[c, pl.ds(s * ROWS_PER_T, ROWS_PER_T)],
    )


@functools.partial(
    pl.kernel,
    out_type=jax.ShapeDtypeStruct((NC, N, D), jnp.float32),
    mesh=_mesh,
    scratch_types=[
        pltpu.VMEM_SHARED((N, D), jnp.float32),
        pltpu.VMEM((NCH_AGG, CHUNK), jnp.int32),
        pltpu.VMEM((NCH_AGG, CHUNK), jnp.int32),
        pltpu.VMEM((CHUNK, D), jnp.float32),
        pltpu.VMEM((CHUNK, D), jnp.float32),
        pltpu.VMEM((CHUNK, D), jnp.float32),
    ],
)
def _agg_kernel(node_hbm, ef_hbm, src_hbm, dst_hbm, part_hbm,
                acc, src_v, dst_v, gbuf, ebuf, zero_v):
    c = lax.axis_index("c")
    s = lax.axis_index("s")
    w = s * NC + c

    _zero_rows(zero_v, CHUNK)
    for k in range(ROWS_PER_T // CHUNK):
        pltpu.sync_copy(zero_v, acc.at[pl.ds(s * ROWS_PER_T + k * CHUNK, CHUNK)])
    plsc.subcore_barrier()

    pltpu.sync_copy(src_hbm.at[w], src_v)
    pltpu.sync_copy(dst_hbm.at[w], dst_v)

    def body(j, _):
        pltpu.sync_copy(node_hbm.at[src_v.at[j]], gbuf)
        pltpu.sync_copy(ef_hbm.at[pl.ds(w * EDGES_PER_W + j * CHUNK, CHUNK)], ebuf)
        pltpu.sync_copy(gbuf, acc.at[dst_v.at[j]], add=True)
        pltpu.sync_copy(ebuf, acc.at[dst_v.at[j]], add=True)
        return 0

    lax.fori_loop(0, NCH_AGG, body, 0)
    plsc.subcore_barrier()

    pltpu.sync_copy(
        acc.at[pl.ds(s * ROWS_PER_T, ROWS_PER_T)],
        part_hbm.at[c, pl.ds(s * ROWS_PER_T, ROWS_PER_T)],
    )


ROWB = 1000


def _proj_body(x_ref, w_ref, cnt_ref, o_ref):
    y = jnp.dot(x_ref[...], w_ref[0], preferred_element_type=jnp.float32,
                precision=lax.Precision.HIGHEST)
    scale = lax.rsqrt(jnp.maximum(cnt_ref[...], 1.0))
    o_ref[...] = y * scale


def _fin_body(p0_ref, p1_ref, cnt_ref, o_ref):
    scale = lax.rsqrt(jnp.maximum(cnt_ref[...], 1.0))
    o_ref[...] = (p0_ref[...] + p1_ref[...]) * scale


def kernel(u_f, v_f, e_f, edge_index, u_w, v_w):
    src = edge_index[0]
    dst = edge_index[1]

    idx_all = jnp.stack([src, dst]).reshape(2, NS, NCH_HIST, CHUNK)
    hist = _hist_kernel(idx_all)
    out_cnt = hist[0, :, 0:1]
    in_cnt = hist[1, :, 0:1]

    x = jnp.concatenate([u_f, v_f], axis=0)
    w = jnp.stack([u_w, v_w])
    node_f = pl.pallas_call(
        _proj_body,
        grid=(N // ROWB,),
        in_specs=[
            pl.BlockSpec((ROWB, D), lambda i: (i, 0)),
            pl.BlockSpec((1, D, D), lambda i: (i // 5, 0, 0)),
            pl.BlockSpec((ROWB, 1), lambda i: (i, 0)),
        ],
        out_specs=pl.BlockSpec((ROWB, D), lambda i: (i, 0)),
        out_shape=jax.ShapeDtypeStruct((N, D), jnp.float32),
    )(x, w, out_cnt)

    src_r = src.reshape(NW, NCH_AGG, CHUNK)
    dst_r = dst.reshape(NW, NCH_AGG, CHUNK)
    partial = _agg_kernel(node_f, e_f, src_r, dst_r)

    rst = pl.pallas_call(
        _fin_body,
        grid=(N // ROWB,),
        in_specs=[
            pl.BlockSpec((ROWB, D), lambda i: (i, 0)),
            pl.BlockSpec((ROWB, D), lambda i: (i, 0)),
            pl.BlockSpec((ROWB, 1), lambda i: (i, 0)),
        ],
        out_specs=pl.BlockSpec((ROWB, D), lambda i: (i, 0)),
        out_shape=jax.ShapeDtypeStruct((N, D), jnp.float32),
    )(partial[0], partial[1], in_cnt)
    return rst


# speedup vs baseline: 2.5955x; 2.5955x over previous
import functools

import jax
import jax.numpy as jnp
from jax import lax
from jax.experimental import pallas as pl
from jax.experimental.pallas import tpu as pltpu
from jax.experimental.pallas import tpu_sc as plsc

N = 10000
NPAD = 10240
E = 320000
D = 128

NC = 2
NS = 16
NW = NC * NS

CHUNK = 125
EDGES_PER_W = E // NW
NCH_AGG = EDGES_PER_W // CHUNK
EDGES_PER_T = E // NS
NCH_HIST = EDGES_PER_T // CHUNK
ROWS_PER_T = NPAD // NS

_mesh = plsc.VectorSubcoreMesh(core_axis_name="c", subcore_axis_name="s")


def _zero_rows(buf, nrows):
    ncol = buf.shape[1] // 16

    def body(t, _):
        r = t // ncol
        col = t % ncol
        buf[r, pl.ds(col * 16, 16)] = jnp.zeros((16,), jnp.float32)
        return 0

    lax.fori_loop(0, nrows * ncol, body, 0)


@functools.partial(
    pl.kernel,
    out_type=jax.ShapeDtypeStruct((NC, NS, 2, NPAD), jnp.float32),
    mesh=_mesh,
    compiler_params=pltpu.CompilerParams(needs_layout_passes=False),
    scratch_types=[
        pltpu.VMEM((EDGES_PER_W,), jnp.int32),
        pltpu.VMEM((EDGES_PER_W,), jnp.int32),
        pltpu.VMEM((NPAD,), jnp.float32),
        pltpu.VMEM((NPAD,), jnp.float32),
    ],
)
def _hist_kernel(src_hbm, dst_hbm, hist_hbm, sidx, didx, cnt_s, cnt_d):
    c = lax.axis_index("c")
    s = lax.axis_index("s")
    w = s * NC + c

    def z(t, _):
        cnt_s[pl.ds(t * 16, 16)] = jnp.zeros((16,), jnp.float32)
        cnt_d[pl.ds(t * 16, 16)] = jnp.zeros((16,), jnp.float32)
        return 0

    lax.fori_loop(0, NPAD // 16, z, 0)

    pltpu.sync_copy(src_hbm.at[pl.ds(w * EDGES_PER_W, EDGES_PER_W)], sidx)
    pltpu.sync_copy(dst_hbm.at[pl.ds(w * EDGES_PER_W, EDGES_PER_W)], didx)

    ones = jnp.ones((16,), jnp.float32)

    def body(t, _):
        si = sidx[pl.ds(t * 16, 16)]
        di = didx[pl.ds(t * 16, 16)]
        plsc.addupdate_scatter(cnt_s, [si], ones)
        plsc.addupdate_scatter(cnt_d, [di], ones)
        return 0

    lax.fori_loop(0, EDGES_PER_W // 16, body, 0)

    pltpu.sync_copy(cnt_s, hist_hbm.at[c, s, 0])
    pltpu.sync_copy(cnt_d, hist_hbm.at[c, s, 1])


NCHB = 16
NBLK = NCH_AGG // NCHB


@functools.partial(
    pl.kernel,
    out_type=jax.ShapeDtypeStruct((NC, NS, ROWS_PER_T, D), jnp.float32),
    mesh=_mesh,
    scratch_types=[
        pltpu.VMEM_SHARED((NPAD, D), jnp.float32),
        pltpu.VMEM((NCHB, CHUNK), jnp.int32),
        pltpu.VMEM((NCHB, CHUNK), jnp.int32),
        pltpu.VMEM((CHUNK, D), jnp.float32),
        pltpu.VMEM((CHUNK, D), jnp.float32),
    ],
)
def _agg_kernel(node_hbm, ef_hbm, src_hbm, dst_hbm, part_hbm,
                acc, src_v, dst_v, gbuf, ebuf):
    c = lax.axis_index("c")
    s = lax.axis_index("s")
    w = s * NC + c

    _zero_rows(gbuf, 80)
    for k in range(8):
        pltpu.sync_copy(gbuf.at[pl.ds(0, 80)],
                        acc.at[pl.ds(s * ROWS_PER_T + k * 80, 80)])
    plsc.subcore_barrier()

    def blk(jb, _):
        pltpu.sync_copy(src_hbm.at[w, pl.ds(jb * NCHB, NCHB)], src_v)
        pltpu.sync_copy(dst_hbm.at[w, pl.ds(jb * NCHB, NCHB)], dst_v)

        def body(r, _):
            j = jb * NCHB + r
            pltpu.sync_copy(node_hbm.at[src_v.at[r]], gbuf)
            pltpu.sync_copy(ef_hbm.at[w, j], ebuf)
            pltpu.sync_copy(gbuf, acc.at[dst_v.at[r]], add=True)
            pltpu.sync_copy(ebuf, acc.at[dst_v.at[r]], add=True)
            return 0

        lax.fori_loop(0, NCHB, body, 0)
        return 0

    lax.fori_loop(0, NBLK, blk, 0)
    plsc.subcore_barrier()

    pltpu.sync_copy(acc.at[pl.ds(s * ROWS_PER_T, ROWS_PER_T)], part_hbm.at[c, s])


ROWB = 1000


def _proj_body(x_ref, w_ref, cnt_ref, o_ref):
    y = jnp.dot(x_ref[...], w_ref[0], preferred_element_type=jnp.float32,
                precision=lax.Precision.HIGHEST)
    cnt = jnp.sum(cnt_ref[...], axis=0)
    scale = lax.rsqrt(jnp.maximum(cnt, 1.0))
    o_ref[...] = y * scale


def _fin_body(p0_ref, p1_ref, cnt_ref, o_ref):
    cnt = jnp.sum(cnt_ref[...], axis=0)
    scale = lax.rsqrt(jnp.maximum(cnt, 1.0))
    o_ref[...] = (p0_ref[...] + p1_ref[...]) * scale


def kernel(u_f, v_f, e_f, edge_index, u_w, v_w):
    src = edge_index[0]
    dst = edge_index[1]

    hist = _hist_kernel(src, dst).reshape(NW, 2, NPAD, 1)
    out_cnt = hist[:, 0, :N]
    in_cnt = hist[:, 1, :N]

    x = jnp.concatenate([u_f, v_f], axis=0)
    w = jnp.stack([u_w, v_w])
    node_f = pl.pallas_call(
        _proj_body,
        grid=(N // ROWB,),
        in_specs=[
            pl.BlockSpec((ROWB, D), lambda i: (i, 0)),
            pl.BlockSpec((1, D, D), lambda i: (i // 5, 0, 0)),
            pl.BlockSpec((NW, ROWB, 1), lambda i: (0, i, 0)),
        ],
        out_specs=pl.BlockSpec((ROWB, D), lambda i: (i, 0)),
        out_shape=jax.ShapeDtypeStruct((N, D), jnp.float32),
    )(x, w, out_cnt)

    src_r = src.reshape(NW, NCH_AGG, CHUNK)
    dst_r = dst.reshape(NW, NCH_AGG, CHUNK)
    ef_r = e_f.reshape(NW, NCH_AGG, CHUNK, D)
    partial = _agg_kernel(node_f, ef_r, src_r, dst_r).reshape(NC, NPAD, D)

    rst = pl.pallas_call(
        _fin_body,
        grid=(N // ROWB,),
        in_specs=[
            pl.BlockSpec((ROWB, D), lambda i: (i, 0)),
            pl.BlockSpec((ROWB, D), lambda i: (i, 0)),
            pl.BlockSpec((NW, ROWB, 1), lambda i: (0, i, 0)),
        ],
        out_specs=pl.BlockSpec((ROWB, D), lambda i: (i, 0)),
        out_shape=jax.ShapeDtypeStruct((N, D), jnp.float32),
    )(partial[0, :N], partial[1, :N], in_cnt)
    return rst


# speedup vs baseline: 3.7615x; 1.4492x over previous
import functools

import jax
import jax.numpy as jnp
from jax import lax
from jax.experimental import pallas as pl
from jax.experimental.pallas import tpu as pltpu
from jax.experimental.pallas import tpu_sc as plsc

N = 10000
NPAD = 10240
E = 320000
D = 128

NC = 2
NS = 16
NW = NC * NS

CHUNK = 80
EDGES_PER_W = E // NW
NCH_AGG = EDGES_PER_W // CHUNK
NCHB = 25
NBLK = NCH_AGG // NCHB
EDGES_PER_T = E // NS
ROWS_PER_T = NPAD // NS

_mesh = plsc.VectorSubcoreMesh(core_axis_name="c", subcore_axis_name="s")


def _zero_1d(buf, n):
    def z(t, _):
        buf[pl.ds(t * 16, 16)] = jnp.zeros((16,), jnp.float32)
        return 0

    lax.fori_loop(0, n // 16, z, 0)


@functools.partial(
    pl.kernel,
    out_type=jax.ShapeDtypeStruct((2, NPAD, D), jnp.float32),
    mesh=_mesh,
    compiler_params=pltpu.CompilerParams(needs_layout_passes=False),
    scratch_types=[
        pltpu.VMEM_SHARED((NS, NPAD), jnp.float32),
        pltpu.VMEM((EDGES_PER_T,), jnp.int32),
        pltpu.VMEM((NPAD,), jnp.float32),
        pltpu.VMEM((ROWS_PER_T,), jnp.float32),
        pltpu.VMEM((ROWS_PER_T,), jnp.float32),
        pltpu.VMEM((CHUNK, D), jnp.float32),
    ],
)
def _scale_kernel(src_hbm, dst_hbm, scale_hbm,
                  staging, idx, cnt, tmp, scalebuf, rowbuf):
    c = lax.axis_index("c")
    s = lax.axis_index("s")

    _zero_1d(cnt, NPAD)

    @pl.when(c == 0)
    def _():
        pltpu.sync_copy(src_hbm.at[pl.ds(s * EDGES_PER_T, EDGES_PER_T)], idx)

    @pl.when(c == 1)
    def _():
        pltpu.sync_copy(dst_hbm.at[pl.ds(s * EDGES_PER_T, EDGES_PER_T)], idx)

    ones = jnp.ones((16,), jnp.float32)

    def body(t, _):
        v = idx[pl.ds(t * 16, 16)]
        plsc.addupdate_scatter(cnt, [v], ones)
        return 0

    lax.fori_loop(0, EDGES_PER_T // 16, body, 0)

    pltpu.sync_copy(cnt, staging.at[s])
    plsc.subcore_barrier()

    _zero_1d(scalebuf, ROWS_PER_T)

    def red(k, _):
        pltpu.sync_copy(staging.at[k, pl.ds(s * ROWS_PER_T, ROWS_PER_T)], tmp)

        def add16(j, _):
            scalebuf[pl.ds(j * 16, 16)] = (
                scalebuf[pl.ds(j * 16, 16)] + tmp[pl.ds(j * 16, 16)])
            return 0

        lax.fori_loop(0, ROWS_PER_T // 16, add16, 0)
        return 0

    lax.fori_loop(0, NS, red, 0)

    magic = jnp.full((16,), 0x5F3759DF, dtype=jnp.int32)

    def rs(j, _):
        x = jnp.maximum(scalebuf[pl.ds(j * 16, 16)], 1.0)
        i = plsc.bitcast(x, jnp.int32)
        y = plsc.bitcast(magic - jnp.right_shift(i, 1), jnp.float32)
        for _ in range(3):
            y = y * (1.5 - 0.5 * x * y * y)
        scalebuf[pl.ds(j * 16, 16)] = y
        return 0

    lax.fori_loop(0, ROWS_PER_T // 16, rs, 0)

    def chunk(k, _):
        def grp(g, _):
            vec16 = scalebuf[pl.ds(k * CHUNK + g * 16, 16)]
            for l in range(16):
                vec = jnp.zeros((16,), jnp.float32) + vec16[l]
                for u in range(D // 16):
                    rowbuf[g * 16 + l, pl.ds(u * 16, 16)] = vec
            return 0

        lax.fori_loop(0, CHUNK // 16, grp, 0)
        pltpu.sync_copy(
            rowbuf, scale_hbm.at[c, pl.ds(s * ROWS_PER_T + k * CHUNK, CHUNK)])
        return 0

    lax.fori_loop(0, ROWS_PER_T // CHUNK, chunk, 0)


@functools.partial(
    pl.kernel,
    out_type=jax.ShapeDtypeStruct((NC, NS, ROWS_PER_T, D), jnp.float32),
    mesh=_mesh,
    scratch_types=[
        pltpu.VMEM_SHARED((NPAD, D), jnp.float32),
        pltpu.VMEM((NCHB, CHUNK), jnp.int32),
        pltpu.VMEM((NCHB, CHUNK), jnp.int32),
        pltpu.VMEM((CHUNK, D), jnp.float32),
        pltpu.VMEM((CHUNK, D), jnp.float32),
    ],
)
def _agg_kernel(node_hbm, ef_hbm, src_hbm, dst_hbm, part_hbm,
                acc, src_v, dst_v, gbuf, ebuf):
    c = lax.axis_index("c")
    s = lax.axis_index("s")
    w = s * NC + c

    def zg(t, _):
        gbuf[t // 8, pl.ds((t % 8) * 16, 16)] = jnp.zeros((16,), jnp.float32)
        return 0

    lax.fori_loop(0, CHUNK * 8, zg, 0)
    for k in range(ROWS_PER_T // CHUNK):
        pltpu.sync_copy(gbuf, acc.at[pl.ds(s * ROWS_PER_T + k * CHUNK, CHUNK)])
    plsc.subcore_barrier()

    def blk(jb, _):
        pltpu.sync_copy(src_hbm.at[w, jb], src_v)
        pltpu.sync_copy(dst_hbm.at[w, jb], dst_v)

        def body(r, _):
            j = jb * NCHB + r
            pltpu.sync_copy(node_hbm.at[src_v.at[r]], gbuf)
            pltpu.sync_copy(
                ef_hbm.at[pl.ds(w * EDGES_PER_W + j * CHUNK, CHUNK)], ebuf)
            pltpu.sync_copy(gbuf, acc.at[dst_v.at[r]], add=True)
            pltpu.sync_copy(ebuf, acc.at[dst_v.at[r]], add=True)
            return 0

        lax.fori_loop(0, NCHB, body, 0)
        return 0

    lax.fori_loop(0, NBLK, blk, 0)
    plsc.subcore_barrier()

    pltpu.sync_copy(acc.at[pl.ds(s * ROWS_PER_T, ROWS_PER_T)], part_hbm.at[c, s])


ROWB = 1000


def _proj_body(x_ref, w_ref, sc_ref, o_ref):
    y = jnp.dot(x_ref[...], w_ref[0], preferred_element_type=jnp.float32,
                precision=lax.Precision.HIGHEST)
    o_ref[...] = y * sc_ref[...]


def _fin_body(p0_ref, p1_ref, sc_ref, o_ref):
    o_ref[...] = (p0_ref[...] + p1_ref[...]) * sc_ref[...]


def kernel(u_f, v_f, e_f, edge_index, u_w, v_w):
    src = edge_index[0]
    dst = edge_index[1]

    scale = _scale_kernel(src, dst)

    x = jnp.concatenate([u_f, v_f], axis=0)
    w = jnp.stack([u_w, v_w])
    node_f = pl.pallas_call(
        _proj_body,
        grid=(N // ROWB,),
        in_specs=[
            pl.BlockSpec((ROWB, D), lambda i: (i, 0)),
            pl.BlockSpec((1, D, D), lambda i: (i // 5, 0, 0)),
            pl.BlockSpec((ROWB, D), lambda i: (i, 0)),
        ],
        out_specs=pl.BlockSpec((ROWB, D), lambda i: (i, 0)),
        out_shape=jax.ShapeDtypeStruct((N, D), jnp.float32),
    )(x, w, scale[0])

    src_r = src.reshape(NW, NBLK, NCHB, CHUNK)
    dst_r = dst.reshape(NW, NBLK, NCHB, CHUNK)
    partial = _agg_kernel(node_f, e_f, src_r, dst_r).reshape(NC, NPAD, D)

    rst = pl.pallas_call(
        _fin_body,
        grid=(N // ROWB,),
        in_specs=[
            pl.BlockSpec((ROWB, D), lambda i: (i, 0)),
            pl.BlockSpec((ROWB, D), lambda i: (i, 0)),
            pl.BlockSpec((ROWB, D), lambda i: (i, 0)),
        ],
        out_specs=pl.BlockSpec((ROWB, D), lambda i: (i, 0)),
        out_shape=jax.ShapeDtypeStruct((N, D), jnp.float32),
    )(partial[0, :N], partial[1, :N], scale[1, :N])
    return rst


# speedup vs baseline: 5.6132x; 1.4923x over previous
import functools

import jax
import jax.numpy as jnp
from jax import lax
from jax.experimental import pallas as pl
from jax.experimental.pallas import tpu as pltpu
from jax.experimental.pallas import tpu_sc as plsc

N = 10000
NPAD = 10240
E = 320000
D = 128

NC = 2
NS = 16
NW = NC * NS

CHUNK = 80
EDGES_PER_W = E // NW
NCH_AGG = EDGES_PER_W // CHUNK
NCHB = 25
NBLK = NCH_AGG // NCHB
EDGES_PER_T = E // NS
ROWS_PER_T = NPAD // NS

_mesh = plsc.VectorSubcoreMesh(core_axis_name="c", subcore_axis_name="s")


def _zero_1d(buf, n):
    def z(t, _):
        buf[pl.ds(t * 16, 16)] = jnp.zeros((16,), jnp.float32)
        return 0

    lax.fori_loop(0, n // 16, z, 0)


@functools.partial(
    pl.kernel,
    out_type=jax.ShapeDtypeStruct((2, NPAD, D), jnp.float32),
    mesh=_mesh,
    compiler_params=pltpu.CompilerParams(needs_layout_passes=False),
    scratch_types=[
        pltpu.VMEM_SHARED((NS, NPAD), jnp.float32),
        pltpu.VMEM((EDGES_PER_T,), jnp.int32),
        pltpu.VMEM((NPAD,), jnp.float32),
        pltpu.VMEM((ROWS_PER_T,), jnp.float32),
        pltpu.VMEM((ROWS_PER_T,), jnp.float32),
        pltpu.VMEM((CHUNK, D), jnp.float32),
    ],
)
def _scale_kernel(src_hbm, dst_hbm, scale_hbm,
                  staging, idx, cnt, tmp, scalebuf, rowbuf):
    c = lax.axis_index("c")
    s = lax.axis_index("s")

    _zero_1d(cnt, NPAD)

    @pl.when(c == 0)
    def _():
        pltpu.sync_copy(src_hbm.at[pl.ds(s * EDGES_PER_T, EDGES_PER_T)], idx)

    @pl.when(c == 1)
    def _():
        pltpu.sync_copy(dst_hbm.at[pl.ds(s * EDGES_PER_T, EDGES_PER_T)], idx)

    ones = jnp.ones((16,), jnp.float32)

    def body(t, _):
        v = idx[pl.ds(t * 16, 16)]
        plsc.addupdate_scatter(cnt, [v], ones)
        return 0

    lax.fori_loop(0, EDGES_PER_T // 16, body, 0)

    pltpu.sync_copy(cnt, staging.at[s])
    plsc.subcore_barrier()

    _zero_1d(scalebuf, ROWS_PER_T)

    def red(k, _):
        pltpu.sync_copy(staging.at[k, pl.ds(s * ROWS_PER_T, ROWS_PER_T)], tmp)

        def add16(j, _):
            scalebuf[pl.ds(j * 16, 16)] = (
                scalebuf[pl.ds(j * 16, 16)] + tmp[pl.ds(j * 16, 16)])
            return 0

        lax.fori_loop(0, ROWS_PER_T // 16, add16, 0)
        return 0

    lax.fori_loop(0, NS, red, 0)

    magic = jnp.full((16,), 0x5F3759DF, dtype=jnp.int32)

    def rs(j, _):
        x = jnp.maximum(scalebuf[pl.ds(j * 16, 16)], 1.0)
        i = plsc.bitcast(x, jnp.int32)
        y = plsc.bitcast(magic - jnp.right_shift(i, 1), jnp.float32)
        for _ in range(3):
            y = y * (1.5 - 0.5 * x * y * y)
        scalebuf[pl.ds(j * 16, 16)] = y
        return 0

    lax.fori_loop(0, ROWS_PER_T // 16, rs, 0)

    def chunk(k, _):
        def grp(g, _):
            vec16 = scalebuf[pl.ds(k * CHUNK + g * 16, 16)]
            for l in range(16):
                vec = jnp.zeros((16,), jnp.float32) + vec16[l]
                for u in range(D // 16):
                    rowbuf[g * 16 + l, pl.ds(u * 16, 16)] = vec
            return 0

        lax.fori_loop(0, CHUNK // 16, grp, 0)
        pltpu.sync_copy(
            rowbuf, scale_hbm.at[c, pl.ds(s * ROWS_PER_T + k * CHUNK, CHUNK)])
        return 0

    lax.fori_loop(0, ROWS_PER_T // CHUNK, chunk, 0)


@functools.partial(
    pl.kernel,
    out_type=jax.ShapeDtypeStruct((NC, NS, ROWS_PER_T, D), jnp.float32),
    mesh=_mesh,
    scratch_types=[
        pltpu.VMEM_SHARED((NPAD, D), jnp.float32),
        pltpu.VMEM((NCHB, CHUNK), jnp.int32),
        pltpu.VMEM((NCHB, CHUNK), jnp.int32),
        pltpu.VMEM((CHUNK, D), jnp.float32),
        pltpu.VMEM((CHUNK, D), jnp.float32),
        pltpu.VMEM((CHUNK, D), jnp.float32),
        pltpu.VMEM((CHUNK, D), jnp.float32),
        pltpu.SemaphoreType.DMA,
        pltpu.SemaphoreType.DMA,
        pltpu.SemaphoreType.DMA,
        pltpu.SemaphoreType.DMA,
    ],
)
def _agg_kernel(node_hbm, ef_hbm, src_hbm, dst_hbm, part_hbm,
                acc, src_v, dst_v, g0, g1, e0, e1, si0, si1, ss0, ss1):
    c = lax.axis_index("c")
    s = lax.axis_index("s")
    w = s * NC + c
    ebase = w * EDGES_PER_W

    def zg(t, _):
        g0[t // 8, pl.ds((t % 8) * 16, 16)] = jnp.zeros((16,), jnp.float32)
        return 0

    lax.fori_loop(0, CHUNK * 8, zg, 0)
    for k in range(ROWS_PER_T // CHUNK):
        pltpu.sync_copy(g0, acc.at[pl.ds(s * ROWS_PER_T + k * CHUNK, CHUNK)])
    plsc.subcore_barrier()

    def start_in(r, jblk, gb, eb, sem):
        pltpu.async_copy(node_hbm.at[src_v.at[r]], gb, sem)
        pltpu.async_copy(
            ef_hbm.at[pl.ds(ebase + (jblk * NCHB + r) * CHUNK, CHUNK)], eb, sem)

    def wait_in(r, jblk, gb, eb, sem):
        pltpu.make_async_copy(node_hbm.at[src_v.at[r]], gb, sem).wait()
        pltpu.make_async_copy(
            ef_hbm.at[pl.ds(ebase + (jblk * NCHB + r) * CHUNK, CHUNK)],
            eb, sem).wait()

    def start_scat(r, gb, eb, sem):
        pltpu.async_copy(gb, acc.at[dst_v.at[r]], sem, add=True)
        pltpu.async_copy(eb, acc.at[dst_v.at[r]], sem, add=True)

    def wait_scat(r, gb, eb, sem):
        pltpu.make_async_copy(gb, acc.at[dst_v.at[r]], sem).wait()
        pltpu.make_async_copy(eb, acc.at[dst_v.at[r]], sem).wait()

    def blk(jb, _):
        pltpu.sync_copy(src_hbm.at[w, jb], src_v)
        pltpu.sync_copy(dst_hbm.at[w, jb], dst_v)

        pltpu.sync_copy(node_hbm.at[src_v.at[0]], g0)
        pltpu.sync_copy(ef_hbm.at[pl.ds(ebase + jb * NCHB * CHUNK, CHUNK)], e0)
        pltpu.sync_copy(g0, acc.at[dst_v.at[0]], add=True)
        pltpu.sync_copy(e0, acc.at[dst_v.at[0]], add=True)

        start_in(1, jb, g0, e0, si0)

        def pair(p, _):
            r0 = 1 + 2 * p
            r1 = 2 + 2 * p
            wait_in(r0, jb, g0, e0, si0)
            start_scat(r0, g0, e0, ss0)
            start_in(r1, jb, g1, e1, si1)
            wait_in(r1, jb, g1, e1, si1)
            start_scat(r1, g1, e1, ss1)
            wait_scat(r0, g0, e0, ss0)

            @pl.when(p < (NCHB - 1) // 2 - 1)
            def _():
                start_in(r0 + 2, jb, g0, e0, si0)

            wait_scat(r1, g1, e1, ss1)
            return 0

        lax.fori_loop(0, (NCHB - 1) // 2, pair, 0)
        return 0

    lax.fori_loop(0, NBLK, blk, 0)
    plsc.subcore_barrier()

    pltpu.sync_copy(acc.at[pl.ds(s * ROWS_PER_T, ROWS_PER_T)], part_hbm.at[c, s])


ROWB = 1000


def _proj_body(x_ref, w_ref, sc_ref, o_ref):
    y = jnp.dot(x_ref[...], w_ref[0], preferred_element_type=jnp.float32,
                precision=lax.Precision.HIGHEST)
    o_ref[...] = y * sc_ref[...]


def _fin_body(p0_ref, p1_ref, sc_ref, o_ref):
    o_ref[...] = (p0_ref[...] + p1_ref[...]) * sc_ref[...]


def kernel(u_f, v_f, e_f, edge_index, u_w, v_w):
    src = edge_index[0]
    dst = edge_index[1]

    scale = _scale_kernel(src, dst)

    x = jnp.concatenate([u_f, v_f], axis=0)
    w = jnp.stack([u_w, v_w])
    node_f = pl.pallas_call(
        _proj_body,
        grid=(N // ROWB,),
        in_specs=[
            pl.BlockSpec((ROWB, D), lambda i: (i, 0)),
            pl.BlockSpec((1, D, D), lambda i: (i // 5, 0, 0)),
            pl.BlockSpec((ROWB, D), lambda i: (i, 0)),
        ],
        out_specs=pl.BlockSpec((ROWB, D), lambda i: (i, 0)),
        out_shape=jax.ShapeDtypeStruct((N, D), jnp.float32),
    )(x, w, scale[0])

    src_r = src.reshape(NW, NBLK, NCHB, CHUNK)
    dst_r = dst.reshape(NW, NBLK, NCHB, CHUNK)
    partial = _agg_kernel(node_f, e_f, src_r, dst_r).reshape(NC, NPAD, D)

    rst = pl.pallas_call(
        _fin_body,
        grid=(N // ROWB,),
        in_specs=[
            pl.BlockSpec((ROWB, D), lambda i: (i, 0)),
            pl.BlockSpec((ROWB, D), lambda i: (i, 0)),
            pl.BlockSpec((ROWB, D), lambda i: (i, 0)),
        ],
        out_specs=pl.BlockSpec((ROWB, D), lambda i: (i, 0)),
        out_shape=jax.ShapeDtypeStruct((N, D), jnp.float32),
    )(partial[0, :N], partial[1, :N], scale[1, :N])
    return rst


# speedup vs baseline: 5.6620x; 1.0087x over previous
import functools

import jax
import jax.numpy as jnp
from jax import lax
from jax.experimental import pallas as pl
from jax.experimental.pallas import tpu as pltpu
from jax.experimental.pallas import tpu_sc as plsc

N = 10000
NPAD = 10240
E = 320000
D = 128

NC = 2
NS = 16
NW = NC * NS

CHUNK = 80
EDGES_PER_W = E // NW
NCH_AGG = EDGES_PER_W // CHUNK
NCHB = 25
NBLK = NCH_AGG // NCHB
EDGES_PER_T = E // NS
ROWS_PER_T = NPAD // NS

_mesh = plsc.VectorSubcoreMesh(core_axis_name="c", subcore_axis_name="s")


def _zero_1d(buf, n):
    def z(t, _):
        buf[pl.ds(t * 32, 16)] = jnp.zeros((16,), jnp.float32)
        buf[pl.ds(t * 32 + 16, 16)] = jnp.zeros((16,), jnp.float32)
        return 0

    lax.fori_loop(0, n // 32, z, 0)


@functools.partial(
    pl.kernel,
    out_type=jax.ShapeDtypeStruct((2, NPAD, D), jnp.float32),
    mesh=_mesh,
    compiler_params=pltpu.CompilerParams(needs_layout_passes=False),
    scratch_types=[
        pltpu.VMEM_SHARED((NS, NPAD), jnp.float32),
        pltpu.VMEM((EDGES_PER_T,), jnp.int32),
        pltpu.VMEM((NPAD,), jnp.float32),
        pltpu.VMEM((ROWS_PER_T,), jnp.float32),
        pltpu.VMEM((ROWS_PER_T,), jnp.float32),
        pltpu.VMEM((CHUNK, D), jnp.float32),
    ],
)
def _scale_kernel(src_hbm, dst_hbm, scale_hbm,
                  staging, idx, cnt, tmp, scalebuf, rowbuf):
    c = lax.axis_index("c")
    s = lax.axis_index("s")

    _zero_1d(cnt, NPAD)

    @pl.when(c == 0)
    def _():
        pltpu.sync_copy(src_hbm.at[pl.ds(s * EDGES_PER_T, EDGES_PER_T)], idx)

    @pl.when(c == 1)
    def _():
        pltpu.sync_copy(dst_hbm.at[pl.ds(s * EDGES_PER_T, EDGES_PER_T)], idx)

    ones = jnp.ones((16,), jnp.float32)

    def body(t, _):
        plsc.addupdate_scatter(cnt, [idx[pl.ds(t * 32, 16)]], ones)
        plsc.addupdate_scatter(cnt, [idx[pl.ds(t * 32 + 16, 16)]], ones)
        return 0

    lax.fori_loop(0, EDGES_PER_T // 32, body, 0)

    pltpu.sync_copy(cnt, staging.at[s])
    plsc.subcore_barrier()

    _zero_1d(scalebuf, ROWS_PER_T)

    def red(k, _):
        pltpu.sync_copy(staging.at[k, pl.ds(s * ROWS_PER_T, ROWS_PER_T)], tmp)

        def add16(j, _):
            scalebuf[pl.ds(j * 16, 16)] = (
                scalebuf[pl.ds(j * 16, 16)] + tmp[pl.ds(j * 16, 16)])
            return 0

        lax.fori_loop(0, ROWS_PER_T // 16, add16, 0)
        return 0

    lax.fori_loop(0, NS, red, 0)

    magic = jnp.full((16,), 0x5F3759DF, dtype=jnp.int32)

    def rs(j, _):
        x = jnp.maximum(scalebuf[pl.ds(j * 16, 16)], 1.0)
        i = plsc.bitcast(x, jnp.int32)
        y = plsc.bitcast(magic - jnp.right_shift(i, 1), jnp.float32)
        for _ in range(3):
            y = y * (1.5 - 0.5 * x * y * y)
        scalebuf[pl.ds(j * 16, 16)] = y
        return 0

    lax.fori_loop(0, ROWS_PER_T // 16, rs, 0)

    def chunk(k, _):
        def grp(g, _):
            vec16 = scalebuf[pl.ds(k * CHUNK + g * 16, 16)]
            for l in range(16):
                vec = jnp.zeros((16,), jnp.float32) + vec16[l]
                for u in range(D // 16):
                    rowbuf[g * 16 + l, pl.ds(u * 16, 16)] = vec
            return 0

        lax.fori_loop(0, CHUNK // 16, grp, 0)
        pltpu.sync_copy(
            rowbuf, scale_hbm.at[c, pl.ds(s * ROWS_PER_T + k * CHUNK, CHUNK)])
        return 0

    lax.fori_loop(0, ROWS_PER_T // CHUNK, chunk, 0)


@functools.partial(
    pl.kernel,
    out_type=jax.ShapeDtypeStruct((NC, NPAD, D), jnp.float32),
    mesh=_mesh,
    scratch_types=[
        pltpu.VMEM_SHARED((NPAD, D), jnp.float32),
        pltpu.VMEM((NCHB, CHUNK), jnp.int32),
        pltpu.VMEM((NCHB, CHUNK), jnp.int32),
        pltpu.VMEM((CHUNK, D), jnp.float32),
        pltpu.VMEM((CHUNK, D), jnp.float32),
        pltpu.VMEM((CHUNK, D), jnp.float32),
        pltpu.VMEM((CHUNK, D), jnp.float32),
        pltpu.SemaphoreType.DMA,
        pltpu.SemaphoreType.DMA,
        pltpu.SemaphoreType.DMA,
        pltpu.SemaphoreType.DMA,
    ],
)
def _agg_kernel(node_hbm, ef_hbm, src_hbm, dst_hbm, part_hbm,
                acc, src_v, dst_v, g0, g1, e0, e1, si0, si1, ss0, ss1):
    c = lax.axis_index("c")
    s = lax.axis_index("s")
    w = s * NC + c
    ebase = w * EDGES_PER_W

    def zg(t, _):
        g0[t // 8, pl.ds((t % 8) * 16, 16)] = jnp.zeros((16,), jnp.float32)
        return 0

    lax.fori_loop(0, CHUNK * 8, zg, 0)
    for k in range(ROWS_PER_T // CHUNK):
        pltpu.sync_copy(g0, acc.at[pl.ds(s * ROWS_PER_T + k * CHUNK, CHUNK)])
    plsc.subcore_barrier()

    def start_in(r, jblk, gb, eb, sem):
        pltpu.async_copy(node_hbm.at[src_v.at[r]], gb, sem)
        pltpu.async_copy(
            ef_hbm.at[pl.ds(ebase + (jblk * NCHB + r) * CHUNK, CHUNK)], eb, sem)

    def wait_in(r, jblk, gb, eb, sem):
        pltpu.make_async_copy(node_hbm.at[src_v.at[r]], gb, sem).wait()
        pltpu.make_async_copy(
            ef_hbm.at[pl.ds(ebase + (jblk * NCHB + r) * CHUNK, CHUNK)],
            eb, sem).wait()

    def start_scat(r, gb, eb, sem):
        pltpu.async_copy(gb, acc.at[dst_v.at[r]], sem, add=True)
        pltpu.async_copy(eb, acc.at[dst_v.at[r]], sem, add=True)

    def wait_scat(r, gb, eb, sem):
        pltpu.make_async_copy(gb, acc.at[dst_v.at[r]], sem).wait()
        pltpu.make_async_copy(eb, acc.at[dst_v.at[r]], sem).wait()

    def blk(jb, _):
        pltpu.sync_copy(src_hbm.at[w, jb], src_v)
        pltpu.sync_copy(dst_hbm.at[w, jb], dst_v)

        pltpu.sync_copy(node_hbm.at[src_v.at[0]], g0)
        pltpu.sync_copy(ef_hbm.at[pl.ds(ebase + jb * NCHB * CHUNK, CHUNK)], e0)
        pltpu.sync_copy(g0, acc.at[dst_v.at[0]], add=True)
        pltpu.sync_copy(e0, acc.at[dst_v.at[0]], add=True)

        start_in(1, jb, g0, e0, si0)

        def pair(p, _):
            r0 = 1 + 2 * p
            r1 = 2 + 2 * p
            wait_in(r0, jb, g0, e0, si0)
            start_scat(r0, g0, e0, ss0)
            start_in(r1, jb, g1, e1, si1)
            wait_in(r1, jb, g1, e1, si1)
            start_scat(r1, g1, e1, ss1)
            wait_scat(r0, g0, e0, ss0)

            @pl.when(p < (NCHB - 1) // 2 - 1)
            def _():
                start_in(r0 + 2, jb, g0, e0, si0)

            wait_scat(r1, g1, e1, ss1)
            return 0

        lax.fori_loop(0, (NCHB - 1) // 2, pair, 0)
        return 0

    lax.fori_loop(0, NBLK, blk, 0)
    plsc.subcore_barrier()

    pltpu.sync_copy(acc.at[pl.ds(s * ROWS_PER_T, ROWS_PER_T)],
                    part_hbm.at[c, pl.ds(s * ROWS_PER_T, ROWS_PER_T)])


ROWB = 1000


def _proj_body(x_ref, w_ref, sc_ref, o_ref):
    y = jnp.dot(x_ref[...], w_ref[0], preferred_element_type=jnp.float32,
                precision=lax.Precision.HIGHEST)
    o_ref[...] = y * sc_ref[...]


def _fin_body(p0_ref, p1_ref, sc_ref, o_ref):
    o_ref[...] = (p0_ref[...] + p1_ref[...]) * sc_ref[...]


def kernel(u_f, v_f, e_f, edge_index, u_w, v_w):
    src = edge_index[0]
    dst = edge_index[1]

    scale = _scale_kernel(src, dst)

    x = jnp.concatenate([u_f, v_f], axis=0)
    w = jnp.stack([u_w, v_w])
    node_f = pl.pallas_call(
        _proj_body,
        grid=(N // ROWB,),
        in_specs=[
            pl.BlockSpec((ROWB, D), lambda i: (i, 0)),
            pl.BlockSpec((1, D, D), lambda i: (i // 5, 0, 0)),
            pl.BlockSpec((ROWB, D), lambda i: (i, 0)),
        ],
        out_specs=pl.BlockSpec((ROWB, D), lambda i: (i, 0)),
        out_shape=jax.ShapeDtypeStruct((N, D), jnp.float32),
    )(x, w, scale[0])

    src_r = src.reshape(NW, NBLK, NCHB, CHUNK)
    dst_r = dst.reshape(NW, NBLK, NCHB, CHUNK)
    partial = _agg_kernel(node_f, e_f, src_r, dst_r)

    rst = pl.pallas_call(
        _fin_body,
        grid=(N // ROWB,),
        in_specs=[
            pl.BlockSpec((ROWB, D), lambda i: (i, 0)),
            pl.BlockSpec((ROWB, D), lambda i: (i, 0)),
            pl.BlockSpec((ROWB, D), lambda i: (i, 0)),
        ],
        out_specs=pl.BlockSpec((ROWB, D), lambda i: (i, 0)),
        out_shape=jax.ShapeDtypeStruct((N, D), jnp.float32),
    )(partial[0], partial[1], scale[1])
    return rst
